# baseline (speedup 1.0000x reference)
import functools

import jax
import jax.numpy as jnp
from jax import lax
from jax.experimental import pallas as pl
from jax.experimental.pallas import tpu as pltpu
from jax.experimental.pallas import tpu_sc as plsc

N = 10000
D = 128
NC = 2
NS = 16
NW = NC * NS
CH = 128
N_ACC = 10240
STRIPE = N_ACC // NS

_ROW_BLK = 1000


def _scale_body(x_ref, n_ref, y_ref):
    y_ref[...] = x_ref[...] * n_ref[...]


def _scale(x, norm_col):
    grid = (N // _ROW_BLK,)
    return pl.pallas_call(
        _scale_body,
        grid=grid,
        in_specs=[
            pl.BlockSpec((_ROW_BLK, D), lambda i: (i, 0)),
            pl.BlockSpec((_ROW_BLK, 1), lambda i: (i, 0)),
        ],
        out_specs=pl.BlockSpec((_ROW_BLK, D), lambda i: (i, 0)),
        out_shape=jax.ShapeDtypeStruct((N, D), jnp.float32),
    )(x, norm_col)


def _epilogue_body(g_ref, n_ref, x_ref, w1_ref, w2_ref, b_ref, o_ref):
    g = g_ref[0] + g_ref[1]
    f1 = n_ref[...] * g
    f2 = x_ref[...] * f1
    acc = lax.dot_general(f1, w1_ref[...], (((1,), (1,)), ((), ())),
                          preferred_element_type=jnp.float32)
    acc += lax.dot_general(f2, w2_ref[...], (((1,), (1,)), ((), ())),
                           preferred_element_type=jnp.float32)
    o_ref[...] = acc + b_ref[...]


def _epilogue(partials, norm_col, x, W1_w, W2_w, bias_row):
    grid = (N // _ROW_BLK,)
    return pl.pallas_call(
        _epilogue_body,
        grid=grid,
        in_specs=[
            pl.BlockSpec((2, _ROW_BLK, D), lambda i: (0, i, 0)),
            pl.BlockSpec((_ROW_BLK, 1), lambda i: (i, 0)),
            pl.BlockSpec((_ROW_BLK, D), lambda i: (i, 0)),
            pl.BlockSpec((D, D), lambda i: (0, 0)),
            pl.BlockSpec((D, D), lambda i: (0, 0)),
            pl.BlockSpec((1, D), lambda i: (0, 0)),
        ],
        out_specs=pl.BlockSpec((_ROW_BLK, D), lambda i: (i, 0)),
        out_shape=jax.ShapeDtypeStruct((N, D), jnp.float32),
    )(partials, norm_col, x, W1_w, W2_w, bias_row)


def _make_sc_scatter(cpw, n_passes=2):
    mesh = plsc.VectorSubcoreMesh(core_axis_name="c", subcore_axis_name="s")
    cpg = cpw // n_passes

    @functools.partial(
        pl.kernel,
        out_type=jax.ShapeDtypeStruct((NC, N_ACC, D), jnp.float32),
        mesh=mesh,
        scratch_types=[
            pltpu.VMEM((cpg, CH), jnp.int32),
            pltpu.VMEM((cpg, CH), jnp.int32),
            pltpu.VMEM((CH, D), jnp.float32),
            pltpu.VMEM((CH, D), jnp.float32),
            pltpu.VMEM_SHARED((N_ACC, D), jnp.float32),
            pltpu.SemaphoreType.DMA,
            pltpu.SemaphoreType.DMA,
            pltpu.SemaphoreType.DMA,
            pltpu.SemaphoreType.DMA,
        ],
    )
    def sc_scatter(y_hbm, srci_hbm, dsti_hbm, zer_hbm, out_hbm,
                   src_v, dst_v, buf0, buf1, acc_sh, sem0, sem1, ssem0, ssem1):
        c = lax.axis_index("c")
        s = lax.axis_index("s")
        w = s * NC + c

        pltpu.sync_copy(zer_hbm, buf0)

        @pl.loop(0, STRIPE // CH)
        def _(i):
            pltpu.async_copy(buf0, acc_sh.at[pl.ds(s * STRIPE + i * CH, CH)],
                             sem1)

        @pl.loop(0, STRIPE // CH)
        def _(i):
            pltpu.make_async_copy(
                buf0, acc_sh.at[pl.ds(s * STRIPE + i * CH, CH)], sem1).wait()

        plsc.subcore_barrier()

        for p in range(n_passes):
            pltpu.sync_copy(srci_hbm.at[w].at[pl.ds(p * cpg, cpg)], src_v)
            pltpu.sync_copy(dsti_hbm.at[w].at[pl.ds(p * cpg, cpg)], dst_v)

            pltpu.async_copy(y_hbm.at[src_v.at[0]], buf0, sem0)

            @pl.loop(0, cpg, step=2)
            def _(j):
                @pl.when(j > 0)
                def _():
                    pltpu.make_async_copy(
                        buf1, acc_sh.at[dst_v.at[j - 1]], ssem1).wait()

                pltpu.async_copy(y_hbm.at[src_v.at[j + 1]], buf1, sem1)
                pltpu.make_async_copy(y_hbm.at[src_v.at[j]], buf0, sem0).wait()
                pltpu.async_copy(buf0, acc_sh.at[dst_v.at[j]], ssem0, add=True)

                pltpu.make_async_copy(
                    buf0, acc_sh.at[dst_v.at[j]], ssem0).wait()

                @pl.when(j + 2 < cpg)
                def _():
                    pltpu.async_copy(y_hbm.at[src_v.at[j + 2]], buf0, sem0)

                pltpu.make_async_copy(y_hbm.at[src_v.at[j + 1]], buf1, sem1).wait()
                pltpu.async_copy(buf1, acc_sh.at[dst_v.at[j + 1]], ssem1,
                                 add=True)

            pltpu.make_async_copy(buf1, acc_sh.at[dst_v.at[cpg - 1]],
                                  ssem1).wait()

        plsc.subcore_barrier()

        @pl.loop(0, STRIPE // CH)
        def _(i):
            base = s * STRIPE + i * CH
            pltpu.async_copy(acc_sh.at[pl.ds(base, CH)],
                             out_hbm.at[c].at[pl.ds(base, CH)], sem0)

        @pl.loop(0, STRIPE // CH)
        def _(i):
            base = s * STRIPE + i * CH
            pltpu.make_async_copy(acc_sh.at[pl.ds(base, CH)],
                                  out_hbm.at[c].at[pl.ds(base, CH)],
                                  sem0).wait()

    return sc_scatter


@jax.jit
def kernel(x, norm, edge_index, W1_w, W1_b, W2_w, W2_b):
    E = edge_index.shape[1]
    epw = -(-E // NW)
    cpw = -(-epw // CH)
    cpw += (-cpw) % 4
    e_pad = NW * cpw * CH

    src = edge_index[0].astype(jnp.int32)
    dst = edge_index[1].astype(jnp.int32)
    pad = e_pad - E
    pad_iota = jnp.arange(pad, dtype=jnp.int32)
    src_p = jnp.concatenate([src, pad_iota % N])
    dst_p = jnp.concatenate([dst, N + pad_iota % (N_ACC - N)])
    src_p = src_p.reshape(NW, cpw, CH)
    dst_p = dst_p.reshape(NW, cpw, CH)

    norm_col = norm[:, None]
    y = _scale(x, norm_col)
    zeros_tile = jnp.zeros((CH, D), jnp.float32)
    partials = _make_sc_scatter(cpw)(y, src_p, dst_p, zeros_tile)
    bias_row = (W1_b + W2_b)[None, :]
    return _epilogue(partials, norm_col, x, W1_w, W2_w, bias_row)


# speedup vs baseline: 1.0406x; 1.0406x over previous
import functools

import jax
import jax.numpy as jnp
from jax import lax
from jax.experimental import pallas as pl
from jax.experimental.pallas import tpu as pltpu
from jax.experimental.pallas import tpu_sc as plsc

N = 10000
D = 128
NC = 2
NS = 16
NW = NC * NS
CH = 128
N_ACC = 10240
STRIPE = N_ACC // NS

_ROW_BLK = 2000


def _scale_body(x_ref, n_ref, y_ref):
    y_ref[...] = x_ref[...] * n_ref[...]


def _scale(x, norm_col):
    grid = (N // _ROW_BLK,)
    return pl.pallas_call(
        _scale_body,
        grid=grid,
        in_specs=[
            pl.BlockSpec((_ROW_BLK, D), lambda i: (i, 0)),
            pl.BlockSpec((_ROW_BLK, 1), lambda i: (i, 0)),
        ],
        out_specs=pl.BlockSpec((_ROW_BLK, D), lambda i: (i, 0)),
        out_shape=jax.ShapeDtypeStruct((N, D), jnp.float32),
    )(x, norm_col)


def _epilogue_body(g_ref, n_ref, x_ref, w1_ref, w2_ref, b_ref, o_ref):
    g = g_ref[0] + g_ref[1]
    f1 = n_ref[...] * g
    f2 = x_ref[...] * f1
    acc = lax.dot_general(f1, w1_ref[...], (((1,), (1,)), ((), ())),
                          preferred_element_type=jnp.float32)
    acc += lax.dot_general(f2, w2_ref[...], (((1,), (1,)), ((), ())),
                           preferred_element_type=jnp.float32)
    o_ref[...] = acc + b_ref[...]


def _epilogue(partials, norm_col, x, W1_w, W2_w, bias_row):
    grid = (N // _ROW_BLK,)
    return pl.pallas_call(
        _epilogue_body,
        grid=grid,
        in_specs=[
            pl.BlockSpec((2, _ROW_BLK, D), lambda i: (0, i, 0)),
            pl.BlockSpec((_ROW_BLK, 1), lambda i: (i, 0)),
            pl.BlockSpec((_ROW_BLK, D), lambda i: (i, 0)),
            pl.BlockSpec((D, D), lambda i: (0, 0)),
            pl.BlockSpec((D, D), lambda i: (0, 0)),
            pl.BlockSpec((1, D), lambda i: (0, 0)),
        ],
        out_specs=pl.BlockSpec((_ROW_BLK, D), lambda i: (i, 0)),
        out_shape=jax.ShapeDtypeStruct((N, D), jnp.float32),
    )(partials, norm_col, x, W1_w, W2_w, bias_row)


def _make_sc_scatter(cpw, n_passes=2):
    mesh = plsc.VectorSubcoreMesh(core_axis_name="c", subcore_axis_name="s")
    cpg = cpw // n_passes

    @functools.partial(
        pl.kernel,
        out_type=jax.ShapeDtypeStruct((NC, N_ACC, D), jnp.float32),
        mesh=mesh,
        scratch_types=[
            pltpu.VMEM((cpg, CH), jnp.int32),
            pltpu.VMEM((cpg, CH), jnp.int32),
            pltpu.VMEM((CH, D), jnp.float32),
            pltpu.VMEM((CH, D), jnp.float32),
            pltpu.VMEM_SHARED((N_ACC, D), jnp.float32),
            pltpu.SemaphoreType.DMA,
            pltpu.SemaphoreType.DMA,
            pltpu.SemaphoreType.DMA,
            pltpu.SemaphoreType.DMA,
        ],
    )
    def sc_scatter(y_hbm, srci_hbm, dsti_hbm, zer_hbm, out_hbm,
                   src_v, dst_v, buf0, buf1, acc_sh, sem0, sem1, ssem0, ssem1):
        c = lax.axis_index("c")
        s = lax.axis_index("s")
        w = s * NC + c

        for p in range(n_passes):
            pltpu.sync_copy(srci_hbm.at[w].at[pl.ds(p * cpg, cpg)], src_v)
            pltpu.sync_copy(dsti_hbm.at[w].at[pl.ds(p * cpg, cpg)], dst_v)
            pltpu.async_copy(y_hbm.at[src_v.at[0]], buf0, sem0)

            if p == 0:
                pltpu.sync_copy(zer_hbm, buf1)

                @pl.loop(0, STRIPE // CH)
                def _(i):
                    pltpu.async_copy(
                        buf1, acc_sh.at[pl.ds(s * STRIPE + i * CH, CH)],
                        ssem1)

                @pl.loop(0, STRIPE // CH)
                def _(i):
                    pltpu.make_async_copy(
                        buf1, acc_sh.at[pl.ds(s * STRIPE + i * CH, CH)],
                        ssem1).wait()

            pltpu.async_copy(y_hbm.at[src_v.at[1]], buf1, sem1)

            if p == 0:
                plsc.subcore_barrier()

            @pl.loop(0, cpg, step=2)
            def _(j):
                @pl.when(j > 0)
                def _():
                    pltpu.make_async_copy(
                        buf1, acc_sh.at[dst_v.at[j - 1]], ssem1).wait()
                    pltpu.async_copy(y_hbm.at[src_v.at[j + 1]], buf1, sem1)

                pltpu.make_async_copy(y_hbm.at[src_v.at[j]], buf0, sem0).wait()
                pltpu.async_copy(buf0, acc_sh.at[dst_v.at[j]], ssem0, add=True)

                pltpu.make_async_copy(
                    buf0, acc_sh.at[dst_v.at[j]], ssem0).wait()

                @pl.when(j + 2 < cpg)
                def _():
                    pltpu.async_copy(y_hbm.at[src_v.at[j + 2]], buf0, sem0)

                pltpu.make_async_copy(y_hbm.at[src_v.at[j + 1]], buf1, sem1).wait()
                pltpu.async_copy(buf1, acc_sh.at[dst_v.at[j + 1]], ssem1,
                                 add=True)

            pltpu.make_async_copy(buf1, acc_sh.at[dst_v.at[cpg - 1]],
                                  ssem1).wait()

        plsc.subcore_barrier()

        @pl.loop(0, STRIPE // CH)
        def _(i):
            base = s * STRIPE + i * CH
            pltpu.async_copy(acc_sh.at[pl.ds(base, CH)],
                             out_hbm.at[c].at[pl.ds(base, CH)], sem0)

        @pl.loop(0, STRIPE // CH)
        def _(i):
            base = s * STRIPE + i * CH
            pltpu.make_async_copy(acc_sh.at[pl.ds(base, CH)],
                                  out_hbm.at[c].at[pl.ds(base, CH)],
                                  sem0).wait()

    return sc_scatter


@jax.jit
def kernel(x, norm, edge_index, W1_w, W1_b, W2_w, W2_b):
    E = edge_index.shape[1]
    epw = -(-E // NW)
    cpw = -(-epw // CH)
    cpw += (-cpw) % 4
    e_pad = NW * cpw * CH

    src = edge_index[0].astype(jnp.int32)
    dst = edge_index[1].astype(jnp.int32)
    pad = e_pad - E
    pad_iota = jnp.arange(pad, dtype=jnp.int32)
    src_p = jnp.concatenate([src, pad_iota % N])
    dst_p = jnp.concatenate([dst, N + pad_iota % (N_ACC - N)])
    src_p = src_p.reshape(NW, cpw, CH)
    dst_p = dst_p.reshape(NW, cpw, CH)

    norm_col = norm[:, None]
    y = _scale(x, norm_col)
    zeros_tile = jnp.zeros((CH, D), jnp.float32)
    partials = _make_sc_scatter(cpw)(y, src_p, dst_p, zeros_tile)
    bias_row = (W1_b + W2_b)[None, :]
    return _epilogue(partials, norm_col, x, W1_w, W2_w, bias_row)


# speedup vs baseline: 1.0430x; 1.0023x over previous
import functools

import jax
import jax.numpy as jnp
from jax import lax
from jax.experimental import pallas as pl
from jax.experimental.pallas import tpu as pltpu
from jax.experimental.pallas import tpu_sc as plsc

N = 10000
D = 128
NC = 2
NS = 16
NW = NC * NS
CH = 128
N_ACC = 10240
STRIPE = N_ACC // NS

_ROW_BLK = 2000


def _scale_body(E, eblk, x_ref, n_ref, e_ref, y_ref, s_ref, d_ref):
    y_ref[...] = x_ref[...] * n_ref[...]
    i = pl.program_id(0)
    eidx = jax.lax.broadcasted_iota(jnp.int32, (1, eblk), 1) + i * eblk
    over = eidx - E
    valid = over < 0
    s_ref[...] = jnp.where(valid, e_ref[0:1, :], over % N)[None]
    d_ref[...] = jnp.where(valid, e_ref[1:2, :], N + over % (N_ACC - N))[None]


def _scale(x, norm_col, edge_index, e_pad):
    grid = (N // _ROW_BLK,)
    E = edge_index.shape[1]
    eblk = e_pad // grid[0]
    return pl.pallas_call(
        functools.partial(_scale_body, E, eblk),
        grid=grid,
        in_specs=[
            pl.BlockSpec((_ROW_BLK, D), lambda i: (i, 0)),
            pl.BlockSpec((_ROW_BLK, 1), lambda i: (i, 0)),
            pl.BlockSpec((2, eblk), lambda i: (0, i)),
        ],
        out_specs=[
            pl.BlockSpec((_ROW_BLK, D), lambda i: (i, 0)),
            pl.BlockSpec((1, 1, eblk), lambda i: (i, 0, 0)),
            pl.BlockSpec((1, 1, eblk), lambda i: (i, 0, 0)),
        ],
        out_shape=[
            jax.ShapeDtypeStruct((N, D), jnp.float32),
            jax.ShapeDtypeStruct((grid[0], 1, eblk), jnp.int32),
            jax.ShapeDtypeStruct((grid[0], 1, eblk), jnp.int32),
        ],
    )(x, norm_col, edge_index)


def _epilogue_body(g_ref, n_ref, x_ref, w1_ref, w2_ref, b_ref, o_ref):
    g = g_ref[0] + g_ref[1]
    f1 = n_ref[...] * g
    f2 = x_ref[...] * f1
    acc = lax.dot_general(f1, w1_ref[...], (((1,), (1,)), ((), ())),
                          preferred_element_type=jnp.float32)
    acc += lax.dot_general(f2, w2_ref[...], (((1,), (1,)), ((), ())),
                           preferred_element_type=jnp.float32)
    o_ref[...] = acc + b_ref[...]


def _epilogue(partials, norm_col, x, W1_w, W2_w, bias_row):
    grid = (N // _ROW_BLK,)
    return pl.pallas_call(
        _epilogue_body,
        grid=grid,
        in_specs=[
            pl.BlockSpec((2, _ROW_BLK, D), lambda i: (0, i, 0)),
            pl.BlockSpec((_ROW_BLK, 1), lambda i: (i, 0)),
            pl.BlockSpec((_ROW_BLK, D), lambda i: (i, 0)),
            pl.BlockSpec((D, D), lambda i: (0, 0)),
            pl.BlockSpec((D, D), lambda i: (0, 0)),
            pl.BlockSpec((1, D), lambda i: (0, 0)),
        ],
        out_specs=pl.BlockSpec((_ROW_BLK, D), lambda i: (i, 0)),
        out_shape=jax.ShapeDtypeStruct((N, D), jnp.float32),
    )(partials, norm_col, x, W1_w, W2_w, bias_row)


def _make_sc_scatter(cpw, n_passes=2):
    mesh = plsc.VectorSubcoreMesh(core_axis_name="c", subcore_axis_name="s")
    cpg = cpw // n_passes

    @functools.partial(
        pl.kernel,
        out_type=jax.ShapeDtypeStruct((NC, N_ACC, D), jnp.float32),
        mesh=mesh,
        scratch_types=[
            pltpu.VMEM((cpg, CH), jnp.int32),
            pltpu.VMEM((cpg, CH), jnp.int32),
            pltpu.VMEM((CH, D), jnp.float32),
            pltpu.VMEM((CH, D), jnp.float32),
            pltpu.VMEM_SHARED((N_ACC, D), jnp.float32),
            pltpu.SemaphoreType.DMA,
            pltpu.SemaphoreType.DMA,
            pltpu.SemaphoreType.DMA,
            pltpu.SemaphoreType.DMA,
        ],
    )
    def sc_scatter(y_hbm, srci_hbm, dsti_hbm, zer_hbm, out_hbm,
                   src_v, dst_v, buf0, buf1, acc_sh, sem0, sem1, ssem0, ssem1):
        c = lax.axis_index("c")
        s = lax.axis_index("s")
        w = s * NC + c

        for p in range(n_passes):
            pltpu.sync_copy(srci_hbm.at[w].at[pl.ds(p * cpg, cpg)], src_v)
            pltpu.sync_copy(dsti_hbm.at[w].at[pl.ds(p * cpg, cpg)], dst_v)
            pltpu.async_copy(y_hbm.at[src_v.at[0]], buf0, sem0)

            if p == 0:
                pltpu.sync_copy(zer_hbm, buf1)

                @pl.loop(0, STRIPE // CH)
                def _(i):
                    pltpu.async_copy(
                        buf1, acc_sh.at[pl.ds(s * STRIPE + i * CH, CH)],
                        ssem1)

                @pl.loop(0, STRIPE // CH)
                def _(i):
                    pltpu.make_async_copy(
                        buf1, acc_sh.at[pl.ds(s * STRIPE + i * CH, CH)],
                        ssem1).wait()

            pltpu.async_copy(y_hbm.at[src_v.at[1]], buf1, sem1)

            if p == 0:
                plsc.subcore_barrier()

            @pl.loop(0, cpg, step=2)
            def _(j):
                @pl.when(j > 0)
                def _():
                    pltpu.make_async_copy(
                        buf1, acc_sh.at[dst_v.at[j - 1]], ssem1).wait()
                    pltpu.async_copy(y_hbm.at[src_v.at[j + 1]], buf1, sem1)

                pltpu.make_async_copy(y_hbm.at[src_v.at[j]], buf0, sem0).wait()
                pltpu.async_copy(buf0, acc_sh.at[dst_v.at[j]], ssem0, add=True)

                pltpu.make_async_copy(
                    buf0, acc_sh.at[dst_v.at[j]], ssem0).wait()

                @pl.when(j + 2 < cpg)
                def _():
                    pltpu.async_copy(y_hbm.at[src_v.at[j + 2]], buf0, sem0)

                pltpu.make_async_copy(y_hbm.at[src_v.at[j + 1]], buf1, sem1).wait()
                pltpu.async_copy(buf1, acc_sh.at[dst_v.at[j + 1]], ssem1,
                                 add=True)

            pltpu.make_async_copy(buf1, acc_sh.at[dst_v.at[cpg - 1]],
                                  ssem1).wait()

        plsc.subcore_barrier()

        @pl.loop(0, STRIPE // CH)
        def _(i):
            base = s * STRIPE + i * CH
            pltpu.async_copy(acc_sh.at[pl.ds(base, CH)],
                             out_hbm.at[c].at[pl.ds(base, CH)], sem0)

        @pl.loop(0, STRIPE // CH)
        def _(i):
            base = s * STRIPE + i * CH
            pltpu.make_async_copy(acc_sh.at[pl.ds(base, CH)],
                                  out_hbm.at[c].at[pl.ds(base, CH)],
                                  sem0).wait()

    return sc_scatter


@jax.jit
def kernel(x, norm, edge_index, W1_w, W1_b, W2_w, W2_b):
    E = edge_index.shape[1]
    epw = -(-E // NW)
    cpw = -(-epw // CH)
    cpw += (-cpw) % 4
    e_pad = NW * cpw * CH

    norm_col = norm[:, None]
    y, src_p, dst_p = _scale(x, norm_col, edge_index.astype(jnp.int32), e_pad)
    src_p = src_p.reshape(NW, cpw, CH)
    dst_p = dst_p.reshape(NW, cpw, CH)
    zeros_tile = jnp.zeros((CH, D), jnp.float32)
    partials = _make_sc_scatter(cpw)(y, src_p, dst_p, zeros_tile)
    bias_row = (W1_b + W2_b)[None, :]
    return _epilogue(partials, norm_col, x, W1_w, W2_w, bias_row)
